# baseline (speedup 1.0000x reference)
import functools

import jax
import jax.numpy as jnp
from jax import lax
from jax.experimental import pallas as pl
from jax.experimental.pallas import tpu as pltpu
from jax.experimental.pallas import tpu_sc as plsc

GAMMA = 12.0
BATCH = 4096
DIM = 128
LANES = 16
NUM_CORES = 2
NUM_SUBCORES = 16
NW = NUM_CORES * NUM_SUBCORES
BPW = BATCH // NW
CHUNKS = DIM // LANES
STRIDE = LANES + 1
NQ = 4
QROWS = BPW // NQ


def _transe_body(entity_hbm, relation_hbm, sample_hbm,
                 out_hbm,
                 sflat, idx_q, rows_q, partials, out_v, sem_s, sem_q):
    wid = lax.axis_index("s") * NUM_CORES + lax.axis_index("c")
    base = wid * BPW

    pltpu.async_copy(sample_hbm.at[pl.ds(base, BPW)], sflat, sem_s).wait()

    lane = lax.iota(jnp.int32, LANES)
    for q in range(NQ):
        for col in range(3):
            colv = jnp.full((LANES,), col, jnp.int32)
            for g in range(QROWS // LANES):
                rows = lane + (q * QROWS + g * LANES)
                idx_q[col][q][pl.ds(g * LANES, LANES)] = plsc.load_gather(
                    sflat, [rows, colv])

    def fire(q):
        return [
            pltpu.async_copy(entity_hbm.at[idx_q[0][q]], rows_q[0][q],
                             sem_q[q]),
            pltpu.async_copy(relation_hbm.at[idx_q[1][q]], rows_q[1][q],
                             sem_q[q]),
            pltpu.async_copy(entity_hbm.at[idx_q[2][q]], rows_q[2][q],
                             sem_q[q]),
        ]

    inflight = {0: fire(0), 1: fire(1)}

    for q in range(NQ):
        for h in inflight.pop(q):
            h.wait()
        if q + 2 < NQ:
            inflight[q + 2] = fire(q + 2)

        h_rows, r_rows, t_rows = rows_q[0][q], rows_q[1][q], rows_q[2][q]

        def row_body(i, carry):
            acc0 = jnp.zeros((LANES,), jnp.float32)
            acc1 = jnp.zeros((LANES,), jnp.float32)
            for c in range(0, CHUNKS, 2):
                hh = h_rows[i, pl.ds(c * LANES, LANES)]
                rr = r_rows[i, pl.ds(c * LANES, LANES)]
                tt = t_rows[i, pl.ds(c * LANES, LANES)]
                acc0 = acc0 + jnp.abs(hh + rr - tt)
                hh = h_rows[i, pl.ds((c + 1) * LANES, LANES)]
                rr = r_rows[i, pl.ds((c + 1) * LANES, LANES)]
                tt = t_rows[i, pl.ds((c + 1) * LANES, LANES)]
                acc1 = acc1 + jnp.abs(hh + rr - tt)
            partials[pl.ds((q * QROWS + i) * STRIDE, LANES)] = acc0 + acc1
            return carry

        lax.fori_loop(0, QROWS, row_body, 0, unroll=2)

        for g in range(QROWS // LANES):
            rows = (lane + (q * QROWS + g * LANES)) * STRIDE
            tot = jnp.zeros((LANES,), jnp.float32)
            for c in range(LANES):
                tot = tot + plsc.load_gather(partials, [rows + c])
            out_v[pl.ds((q * QROWS + g * LANES), LANES)] = GAMMA - tot

    pltpu.sync_copy(out_v, out_hbm.at[pl.ds(base, BPW)])


_transe_sc = functools.partial(
    pl.kernel,
    mesh=plsc.VectorSubcoreMesh(core_axis_name="c", subcore_axis_name="s"),
    out_type=jax.ShapeDtypeStruct((BATCH,), jnp.float32),
    compiler_params=pltpu.CompilerParams(needs_layout_passes=False),
    scratch_types=[
        pltpu.VMEM((BPW, 3), jnp.int32),
        [[pltpu.VMEM((QROWS,), jnp.int32) for _ in range(NQ)]
         for _ in range(3)],
        [[pltpu.VMEM((QROWS, DIM), jnp.float32) for _ in range(NQ)]
         for _ in range(3)],
        pltpu.VMEM((BPW * STRIDE,), jnp.float32),
        pltpu.VMEM((BPW,), jnp.float32),
        pltpu.SemaphoreType.DMA,
        [pltpu.SemaphoreType.DMA for _ in range(NQ)],
    ],
)(_transe_body)


@jax.jit
def kernel(sample, entity_embedding, relation_embedding):
    score = _transe_sc(entity_embedding, relation_embedding, sample)
    return score.reshape(BATCH, 1)


# speedup vs baseline: 1.0701x; 1.0701x over previous
import functools

import jax
import jax.numpy as jnp
from jax import lax
from jax.experimental import pallas as pl
from jax.experimental.pallas import tpu as pltpu
from jax.experimental.pallas import tpu_sc as plsc

GAMMA = 12.0
BATCH = 4096
DIM = 128
LANES = 16
NUM_CORES = 2
NUM_SUBCORES = 16
NW = NUM_CORES * NUM_SUBCORES
BPW = BATCH // NW
CHUNKS = DIM // LANES
STRIDE = LANES + 1
NQ = 4
QROWS = BPW // NQ


def _transe_body(entity_hbm, relation_hbm, hidx_hbm, ridx_hbm, tidx_hbm,
                 out_hbm,
                 idx_q, rows_q, partials, out_v, sem_s, sem_q):
    wid = lax.axis_index("s") * NUM_CORES + lax.axis_index("c")
    base = wid * BPW

    idx_copies = []
    for q in range(NQ):
        for col, src in enumerate((hidx_hbm, ridx_hbm, tidx_hbm)):
            idx_copies.append(pltpu.async_copy(
                src.at[pl.ds(base + q * QROWS, QROWS)], idx_q[col][q],
                sem_s))
    for h in idx_copies:
        h.wait()

    def fire(q):
        return [
            pltpu.async_copy(entity_hbm.at[idx_q[0][q]], rows_q[0][q],
                             sem_q[q]),
            pltpu.async_copy(relation_hbm.at[idx_q[1][q]], rows_q[1][q],
                             sem_q[q]),
            pltpu.async_copy(entity_hbm.at[idx_q[2][q]], rows_q[2][q],
                             sem_q[q]),
        ]

    inflight = {0: fire(0), 1: fire(1)}

    for q in range(NQ):
        for h in inflight.pop(q):
            h.wait()
        if q + 2 < NQ:
            inflight[q + 2] = fire(q + 2)

        h_rows, r_rows, t_rows = rows_q[0][q], rows_q[1][q], rows_q[2][q]

        def row_body(i, carry):
            acc0 = jnp.zeros((LANES,), jnp.float32)
            acc1 = jnp.zeros((LANES,), jnp.float32)
            for c in range(0, CHUNKS, 2):
                hh = h_rows[i, pl.ds(c * LANES, LANES)]
                rr = r_rows[i, pl.ds(c * LANES, LANES)]
                tt = t_rows[i, pl.ds(c * LANES, LANES)]
                acc0 = acc0 + jnp.abs(hh + rr - tt)
                hh = h_rows[i, pl.ds((c + 1) * LANES, LANES)]
                rr = r_rows[i, pl.ds((c + 1) * LANES, LANES)]
                tt = t_rows[i, pl.ds((c + 1) * LANES, LANES)]
                acc1 = acc1 + jnp.abs(hh + rr - tt)
            partials[pl.ds((q * QROWS + i) * STRIDE, LANES)] = acc0 + acc1
            return carry

        lax.fori_loop(0, QROWS, row_body, 0, unroll=2)

    lane = lax.iota(jnp.int32, LANES)

    def tr_body(g, carry):
        rows = (lane + g * LANES) * STRIDE
        tot = jnp.zeros((LANES,), jnp.float32)
        for c in range(LANES):
            tot = tot + plsc.load_gather(partials, [rows + c])
        out_v[pl.ds(g * LANES, LANES)] = GAMMA - tot
        return carry

    lax.fori_loop(0, BPW // LANES, tr_body, 0)

    pltpu.sync_copy(out_v, out_hbm.at[pl.ds(base, BPW)])


_transe_sc = functools.partial(
    pl.kernel,
    mesh=plsc.VectorSubcoreMesh(core_axis_name="c", subcore_axis_name="s"),
    out_type=jax.ShapeDtypeStruct((BATCH,), jnp.float32),
    compiler_params=pltpu.CompilerParams(needs_layout_passes=False),
    scratch_types=[
        [[pltpu.VMEM((QROWS,), jnp.int32) for _ in range(NQ)]
         for _ in range(3)],
        [[pltpu.VMEM((QROWS, DIM), jnp.float32) for _ in range(NQ)]
         for _ in range(3)],
        pltpu.VMEM((BPW * STRIDE,), jnp.float32),
        pltpu.VMEM((BPW,), jnp.float32),
        pltpu.SemaphoreType.DMA,
        [pltpu.SemaphoreType.DMA for _ in range(NQ)],
    ],
)(_transe_body)


@jax.jit
def kernel(sample, entity_embedding, relation_embedding):
    score = _transe_sc(entity_embedding, relation_embedding,
                       sample[:, 0], sample[:, 1], sample[:, 2])
    return score.reshape(BATCH, 1)


# speedup vs baseline: 1.1194x; 1.0461x over previous
import functools

import jax
import jax.numpy as jnp
from jax import lax
from jax.experimental import pallas as pl
from jax.experimental.pallas import tpu as pltpu
from jax.experimental.pallas import tpu_sc as plsc

GAMMA = 12.0
BATCH = 4096
DIM = 128
LANES = 16
NUM_CORES = 2
NUM_SUBCORES = 16
NW = NUM_CORES * NUM_SUBCORES
BPW = BATCH // NW
CHUNKS = DIM // LANES
STRIDE = LANES + 1
NB = 8
BROWS = BPW // NB


def _transe_body(entity_hbm, relation_hbm, hidx_hbm, ridx_hbm, tidx_hbm,
                 out_hbm,
                 hidx_v, ridx_v, tidx_v, h_rows, r_rows, t_rows,
                 partials, out_v, sem_i, sem_a, sem_b):
    wid = lax.axis_index("s") * NUM_CORES + lax.axis_index("c")
    base = wid * BPW

    c0 = pltpu.async_copy(hidx_hbm.at[pl.ds(base, BPW)], hidx_v, sem_i)
    c1 = pltpu.async_copy(ridx_hbm.at[pl.ds(base, BPW)], ridx_v, sem_i)
    c2 = pltpu.async_copy(tidx_hbm.at[pl.ds(base, BPW)], tidx_v, sem_i)
    c0.wait()
    c1.wait()
    c2.wait()

    def block_copies(b, sem):
        s = pl.ds(b * BROWS, BROWS)
        return [
            pltpu.make_async_copy(entity_hbm.at[hidx_v.at[s]],
                                  h_rows.at[s, :], sem),
            pltpu.make_async_copy(relation_hbm.at[ridx_v.at[s]],
                                  r_rows.at[s, :], sem),
            pltpu.make_async_copy(entity_hbm.at[tidx_v.at[s]],
                                  t_rows.at[s, :], sem),
        ]

    def fire(b, sem):
        for c in block_copies(b, sem):
            c.start()

    def drain(b, sem):
        for c in block_copies(b, sem):
            c.wait()

    fire(0, sem_a)
    fire(1, sem_b)

    lane = lax.iota(jnp.int32, LANES)

    def block_body(b, carry):
        even = b % 2 == 0

        @pl.when(even)
        def _():
            drain(b, sem_a)

        @pl.when(jnp.logical_not(even))
        def _():
            drain(b, sem_b)

        @pl.when(jnp.logical_and(even, b < NB - 2))
        def _():
            fire(b + 2, sem_a)

        @pl.when(jnp.logical_and(jnp.logical_not(even), b < NB - 2))
        def _():
            fire(b + 2, sem_b)

        def row_body(i, carry2):
            row = b * BROWS + i
            acc0 = jnp.zeros((LANES,), jnp.float32)
            acc1 = jnp.zeros((LANES,), jnp.float32)
            for c in range(0, CHUNKS, 2):
                hh = h_rows[row, pl.ds(c * LANES, LANES)]
                rr = r_rows[row, pl.ds(c * LANES, LANES)]
                tt = t_rows[row, pl.ds(c * LANES, LANES)]
                acc0 = acc0 + jnp.abs(hh + rr - tt)
                hh = h_rows[row, pl.ds((c + 1) * LANES, LANES)]
                rr = r_rows[row, pl.ds((c + 1) * LANES, LANES)]
                tt = t_rows[row, pl.ds((c + 1) * LANES, LANES)]
                acc1 = acc1 + jnp.abs(hh + rr - tt)
            partials[pl.ds(row * STRIDE, LANES)] = acc0 + acc1
            return carry2

        lax.fori_loop(0, BROWS, row_body, 0, unroll=2)

        rows = (lane + b * BROWS) * STRIDE
        tot = jnp.zeros((LANES,), jnp.float32)
        for c in range(LANES):
            tot = tot + plsc.load_gather(partials, [rows + c])
        out_v[pl.ds(b * BROWS, LANES)] = GAMMA - tot
        return carry

    lax.fori_loop(0, NB, block_body, 0)

    pltpu.sync_copy(out_v, out_hbm.at[pl.ds(base, BPW)])


_transe_sc = functools.partial(
    pl.kernel,
    mesh=plsc.VectorSubcoreMesh(core_axis_name="c", subcore_axis_name="s"),
    out_type=jax.ShapeDtypeStruct((BATCH,), jnp.float32),
    compiler_params=pltpu.CompilerParams(needs_layout_passes=False),
    scratch_types=[
        pltpu.VMEM((BPW,), jnp.int32),
        pltpu.VMEM((BPW,), jnp.int32),
        pltpu.VMEM((BPW,), jnp.int32),
        pltpu.VMEM((BPW, DIM), jnp.float32),
        pltpu.VMEM((BPW, DIM), jnp.float32),
        pltpu.VMEM((BPW, DIM), jnp.float32),
        pltpu.VMEM((BPW * STRIDE,), jnp.float32),
        pltpu.VMEM((BPW,), jnp.float32),
        pltpu.SemaphoreType.DMA,
        pltpu.SemaphoreType.DMA,
        pltpu.SemaphoreType.DMA,
    ],
)(_transe_body)


@jax.jit
def kernel(sample, entity_embedding, relation_embedding):
    score = _transe_sc(entity_embedding, relation_embedding,
                       sample[:, 0], sample[:, 1], sample[:, 2])
    return score.reshape(BATCH, 1)


# speedup vs baseline: 1.1341x; 1.0131x over previous
import functools

import jax
import jax.numpy as jnp
from jax import lax
from jax.experimental import pallas as pl
from jax.experimental.pallas import tpu as pltpu
from jax.experimental.pallas import tpu_sc as plsc

GAMMA = 12.0
BATCH = 4096
DIM = 128
LANES = 16
NUM_CORES = 2
NUM_SUBCORES = 16
NW = NUM_CORES * NUM_SUBCORES
BPW = BATCH // NW
CHUNKS = DIM // LANES
STRIDE = LANES + 1
NB = 8
BROWS = BPW // NB


def _transe_body(entity_hbm, relation_hbm, idx_hbm,
                 out_hbm,
                 idx_v, h_rows, r_rows, t_rows,
                 partials, out_v, sem_i, sem_a, sem_b):
    wid = lax.axis_index("s") * NUM_CORES + lax.axis_index("c")
    base = wid * BPW

    pltpu.async_copy(idx_hbm.at[pl.ds(wid * 3 * BPW, 3 * BPW)], idx_v,
                     sem_i).wait()

    def block_copies(b, sem):
        s = pl.ds(b * BROWS, BROWS)
        return [
            pltpu.make_async_copy(
                entity_hbm.at[idx_v.at[pl.ds(b * BROWS, BROWS)]],
                h_rows.at[s, :], sem),
            pltpu.make_async_copy(
                relation_hbm.at[idx_v.at[pl.ds(BPW + b * BROWS, BROWS)]],
                r_rows.at[s, :], sem),
            pltpu.make_async_copy(
                entity_hbm.at[idx_v.at[pl.ds(2 * BPW + b * BROWS, BROWS)]],
                t_rows.at[s, :], sem),
        ]

    def fire(b, sem):
        for c in block_copies(b, sem):
            c.start()

    def drain(b, sem):
        for c in block_copies(b, sem):
            c.wait()

    fire(0, sem_a)
    fire(1, sem_b)

    lane = lax.iota(jnp.int32, LANES)

    def block_body(b, carry):
        even = b % 2 == 0

        @pl.when(even)
        def _():
            drain(b, sem_a)

        @pl.when(jnp.logical_not(even))
        def _():
            drain(b, sem_b)

        @pl.when(jnp.logical_and(even, b < NB - 2))
        def _():
            fire(b + 2, sem_a)

        @pl.when(jnp.logical_and(jnp.logical_not(even), b < NB - 2))
        def _():
            fire(b + 2, sem_b)

        def row_body(i, carry2):
            row = b * BROWS + i
            acc0 = jnp.zeros((LANES,), jnp.float32)
            acc1 = jnp.zeros((LANES,), jnp.float32)
            for c in range(0, CHUNKS, 2):
                hh = h_rows[row, pl.ds(c * LANES, LANES)]
                rr = r_rows[row, pl.ds(c * LANES, LANES)]
                tt = t_rows[row, pl.ds(c * LANES, LANES)]
                acc0 = acc0 + jnp.abs(hh + rr - tt)
                hh = h_rows[row, pl.ds((c + 1) * LANES, LANES)]
                rr = r_rows[row, pl.ds((c + 1) * LANES, LANES)]
                tt = t_rows[row, pl.ds((c + 1) * LANES, LANES)]
                acc1 = acc1 + jnp.abs(hh + rr - tt)
            partials[pl.ds(row * STRIDE, LANES)] = acc0 + acc1
            return carry2

        lax.fori_loop(0, BROWS, row_body, 0, unroll=2)

        rows = (lane + b * BROWS) * STRIDE
        tot = jnp.zeros((LANES,), jnp.float32)
        for c in range(LANES):
            tot = tot + plsc.load_gather(partials, [rows + c])
        out_v[pl.ds(b * BROWS, LANES)] = GAMMA - tot
        return carry

    lax.fori_loop(0, NB, block_body, 0)

    pltpu.sync_copy(out_v, out_hbm.at[pl.ds(base, BPW)])


_transe_sc = functools.partial(
    pl.kernel,
    mesh=plsc.VectorSubcoreMesh(core_axis_name="c", subcore_axis_name="s"),
    out_type=jax.ShapeDtypeStruct((BATCH,), jnp.float32),
    compiler_params=pltpu.CompilerParams(needs_layout_passes=False),
    scratch_types=[
        pltpu.VMEM((3 * BPW,), jnp.int32),
        pltpu.VMEM((BPW, DIM), jnp.float32),
        pltpu.VMEM((BPW, DIM), jnp.float32),
        pltpu.VMEM((BPW, DIM), jnp.float32),
        pltpu.VMEM((BPW * STRIDE,), jnp.float32),
        pltpu.VMEM((BPW,), jnp.float32),
        pltpu.SemaphoreType.DMA,
        pltpu.SemaphoreType.DMA,
        pltpu.SemaphoreType.DMA,
    ],
)(_transe_body)


@jax.jit
def kernel(sample, entity_embedding, relation_embedding):
    idx_packed = jnp.concatenate(
        [sample[:, 0].reshape(NW, BPW), sample[:, 1].reshape(NW, BPW),
         sample[:, 2].reshape(NW, BPW)], axis=1).reshape(3 * BATCH)
    score = _transe_sc(entity_embedding, relation_embedding, idx_packed)
    return score.reshape(BATCH, 1)
